# Initial kernel scaffold; baseline (speedup 1.0000x reference)
import functools

import jax
import jax.numpy as jnp
from jax import lax
from jax.experimental import pallas as pl
from jax.experimental.pallas import tpu as pltpu
from jax.experimental.pallas import tpu_sc as plsc

_N = 10000
_E = 320000
_R = 3
_D = 128
_C = 16

_CHUNK = 80
_NW = 32
_CPT = _E // (_CHUNK * _NW)
_NCHUNK = _E // _CHUNK
_RPT = _N // 16

_BLK = 1000


def _deg_body(row_hbm, ones_hbm, zeros_hbm, out_hbm, onesv, rowv, acc):
    cid = lax.axis_index("c")
    sid = lax.axis_index("s")
    wid = sid * 2 + cid
    pltpu.sync_copy(ones_hbm, onesv)
    for r in range(_R):
        pltpu.sync_copy(zeros_hbm, acc.at[pl.ds(sid * _RPT, _RPT)])
        plsc.subcore_barrier()
        pltpu.sync_copy(row_hbm.at[r, pl.ds(wid * _CPT, _CPT)], rowv)

        def chunk(j, carry):
            pltpu.sync_copy(onesv, acc.at[rowv.at[j]], add=True)
            return carry

        lax.fori_loop(0, _CPT, chunk, 0)
        plsc.subcore_barrier()
        pltpu.sync_copy(acc.at[pl.ds(sid * _RPT, _RPT)],
                        out_hbm.at[r, cid, pl.ds(sid * _RPT, _RPT)])


def _sc_deg(rows3, ones16, zeros16):
    mesh = plsc.VectorSubcoreMesh(core_axis_name="c", subcore_axis_name="s")
    return pl.kernel(
        _deg_body,
        out_type=jax.ShapeDtypeStruct((_R, 2, _N, 16), jnp.float32),
        mesh=mesh,
        scratch_types=[
            pltpu.VMEM((_CHUNK, 16), jnp.float32),
            pltpu.VMEM((_CPT, _CHUNK), jnp.int32),
            pltpu.VMEM_SHARED((_N, 16), jnp.float32),
        ],
    )(rows3, ones16, zeros16)


def _spmm_body(y_hbm, col_hbm, row_hbm, zeros_hbm, out_hbm,
               colv, rowv, buf, acc, sem):
    cid = lax.axis_index("c")
    sid = lax.axis_index("s")
    wid = sid * 2 + cid
    for r in range(_R):
        pltpu.sync_copy(zeros_hbm, acc.at[pl.ds(sid * _RPT, _RPT)])
        plsc.subcore_barrier()
        pltpu.sync_copy(col_hbm.at[r, pl.ds(wid * _CPT, _CPT)], colv)
        pltpu.sync_copy(row_hbm.at[r, pl.ds(wid * _CPT, _CPT)], rowv)

        def chunk(j, carry):
            pltpu.async_copy(y_hbm.at[colv.at[j]], buf, sem).wait()
            pltpu.sync_copy(buf, acc.at[rowv.at[j]], add=True)
            return carry

        lax.fori_loop(0, _CPT, chunk, 0)
        plsc.subcore_barrier()
        pltpu.sync_copy(acc.at[pl.ds(sid * _RPT, _RPT)],
                        out_hbm.at[r, cid, pl.ds(sid * _RPT, _RPT)])


def _sc_spmm(y_flat, cols3, rows3, zeros128):
    mesh = plsc.VectorSubcoreMesh(core_axis_name="c", subcore_axis_name="s")
    return pl.kernel(
        _spmm_body,
        out_type=jax.ShapeDtypeStruct((_R, 2, _N, _D), jnp.float32),
        mesh=mesh,
        scratch_types=[
            pltpu.VMEM((_CPT, _CHUNK), jnp.int32),
            pltpu.VMEM((_CPT, _CHUNK), jnp.int32),
            pltpu.VMEM((_CHUNK, _D), jnp.float32),
            pltpu.VMEM_SHARED((_N, _D), jnp.float32),
            pltpu.SemaphoreType.DMA,
        ],
    )(y_flat, cols3, rows3, zeros128)


def _prep_body(x_ref, degp_ref, y1_ref, dinv_ref):
    x = x_ref[...]
    dinvs = []
    for r in range(_R):
        deg = degp_ref[r, 0, :, 0:1] + degp_ref[r, 1, :, 0:1]
        dinv = jnp.where(deg > 0, lax.rsqrt(jnp.maximum(deg, 1e-12)), 0.0)
        y1_ref[r] = dinv * x
        dinvs.append(dinv)
    dinv_ref[...] = jnp.concatenate(dinvs, axis=1)


def _tc_prep(X, degp):
    return pl.pallas_call(
        _prep_body,
        grid=(_N // _BLK,),
        in_specs=[
            pl.BlockSpec((_BLK, _D), lambda i: (i, 0)),
            pl.BlockSpec((_R, 2, _BLK, 16), lambda i: (0, 0, i, 0)),
        ],
        out_specs=[
            pl.BlockSpec((_R, _BLK, _D), lambda i: (0, i, 0)),
            pl.BlockSpec((_BLK, _R), lambda i: (i, 0)),
        ],
        out_shape=[
            jax.ShapeDtypeStruct((_R, _N, _D), jnp.float32),
            jax.ShapeDtypeStruct((_N, _R), jnp.float32),
        ],
    )(X, degp)


def _mid_body(p_ref, dinv_ref, w1_ref, y2_ref):
    acc = jnp.zeros((_BLK, _D), jnp.float32)
    for r in range(_R):
        z = dinv_ref[:, r:r + 1] * (p_ref[r, 0] + p_ref[r, 1])
        acc += jax.nn.relu(jnp.dot(z, w1_ref[r],
                                   preferred_element_type=jnp.float32))
    h1 = acc * (1.0 / _R)
    for r in range(_R):
        y2_ref[r] = dinv_ref[:, r:r + 1] * h1


def _tc_mid(p1, dinv, W1):
    return pl.pallas_call(
        _mid_body,
        grid=(_N // _BLK,),
        in_specs=[
            pl.BlockSpec((_R, 2, _BLK, _D), lambda i: (0, 0, i, 0)),
            pl.BlockSpec((_BLK, _R), lambda i: (i, 0)),
            pl.BlockSpec((_R, _D, _D), lambda i: (0, 0, 0)),
        ],
        out_specs=pl.BlockSpec((_R, _BLK, _D), lambda i: (0, i, 0)),
        out_shape=jax.ShapeDtypeStruct((_R, _N, _D), jnp.float32),
    )(p1, dinv, W1)


def _out_body(p_ref, dinv_ref, w2_ref, q_ref, tau_ref, ow_ref, ob_ref,
              logits_ref, alpha_ref):
    hs = []
    ss = []
    q = q_ref[...]
    for r in range(_R):
        z = dinv_ref[:, r:r + 1] * (p_ref[r, 0] + p_ref[r, 1])
        h = jax.nn.relu(jnp.dot(z, w2_ref[r],
                                preferred_element_type=jnp.float32))
        hs.append(h)
        ss.append(jnp.sum(h * q, axis=1, keepdims=True))
    tau_c = jnp.clip(tau_ref[0, 0], 0.5, 5.0)
    m = jnp.maximum(jnp.maximum(ss[0], ss[1]), ss[2])
    es = [jnp.exp((s - m) / tau_c) for s in ss]
    den = es[0] + es[1] + es[2]
    alphas = [e / den for e in es]
    h2 = (hs[0] + hs[1] + hs[2]) * (1.0 / _R)
    for r in range(_R):
        h2 = h2 + alphas[r] * hs[r]
    logits_ref[...] = (jnp.dot(h2, ow_ref[...],
                               preferred_element_type=jnp.float32)
                       + ob_ref[...])
    alpha_ref[...] = jnp.concatenate(alphas, axis=1)


def _tc_out(p2, dinv, W2, q2d, tau2d, out_W, ob2d):
    return pl.pallas_call(
        _out_body,
        grid=(_N // _BLK,),
        in_specs=[
            pl.BlockSpec((_R, 2, _BLK, _D), lambda i: (0, 0, i, 0)),
            pl.BlockSpec((_BLK, _R), lambda i: (i, 0)),
            pl.BlockSpec((_R, _D, _D), lambda i: (0, 0, 0)),
            pl.BlockSpec((1, _D), lambda i: (0, 0)),
            pl.BlockSpec((1, 1), lambda i: (0, 0)),
            pl.BlockSpec((_D, _C), lambda i: (0, 0)),
            pl.BlockSpec((1, _C), lambda i: (0, 0)),
        ],
        out_specs=[
            pl.BlockSpec((_BLK, _C), lambda i: (i, 0)),
            pl.BlockSpec((_BLK, _R), lambda i: (i, 0)),
        ],
        out_shape=[
            jax.ShapeDtypeStruct((_N, _C), jnp.float32),
            jax.ShapeDtypeStruct((_N, _R), jnp.float32),
        ],
    )(p2, dinv, W2, q2d, tau2d, out_W, ob2d)


@jax.jit
def kernel(X, edge_index_r0, edge_index_r1, edge_index_r2, W1, W2,
           att_q, tau, out_W, out_b):
    edges = jnp.stack([edge_index_r0, edge_index_r1, edge_index_r2])
    rows3 = edges[:, 0, :].reshape(_R, _NCHUNK, _CHUNK)
    cols3 = (edges[:, 1, :]
             + (jnp.arange(_R, dtype=jnp.int32) * _N)[:, None]
             ).reshape(_R, _NCHUNK, _CHUNK)

    ones16 = jnp.ones((_CHUNK, 16), jnp.float32)
    zeros16 = jnp.zeros((_RPT, 16), jnp.float32)
    zeros128 = jnp.zeros((_RPT, _D), jnp.float32)

    degp = _sc_deg(rows3, ones16, zeros16)
    y1, dinv = _tc_prep(X, degp)
    p1 = _sc_spmm(y1.reshape(_R * _N, _D), cols3, rows3, zeros128)
    y2 = _tc_mid(p1, dinv, W1)
    p2 = _sc_spmm(y2.reshape(_R * _N, _D), cols3, rows3, zeros128)
    logits, alpha = _tc_out(
        p2, dinv, W2,
        att_q.reshape(1, _D),
        tau.reshape(1, 1),
        out_W,
        out_b.reshape(1, _C),
    )
    return logits, alpha


# speedup vs baseline: 15.2885x; 15.2885x over previous
import functools

import jax
import jax.numpy as jnp
from jax import lax
from jax.experimental import pallas as pl
from jax.experimental.pallas import tpu as pltpu
from jax.experimental.pallas import tpu_sc as plsc

_N = 10000
_E = 320000
_R = 3
_D = 128
_C = 16

_CHUNK = 80
_NW = 32
_CPT = _E // (_CHUNK * _NW)
_NP = 10240
_RPT = _NP // 16

_BLK = 1024


def _deg_body(row_hbm, ones_hbm, zeros_hbm, out_hbm, onesv, rowv, acc):
    cid = lax.axis_index("c")
    sid = lax.axis_index("s")
    wid = sid * 2 + cid
    pltpu.sync_copy(ones_hbm, onesv)
    for r in range(_R):
        pltpu.sync_copy(zeros_hbm, acc.at[pl.ds(sid * _RPT, _RPT)])
        plsc.subcore_barrier()
        pltpu.sync_copy(row_hbm.at[r, wid], rowv)

        def chunk(j, carry):
            pltpu.sync_copy(onesv, acc.at[rowv.at[j]], add=True)
            return carry

        lax.fori_loop(0, _CPT, chunk, 0)
        plsc.subcore_barrier()
        base = pl.multiple_of((r * 2 + cid) * _NP + sid * _RPT, 128)
        pltpu.sync_copy(acc.at[pl.ds(sid * _RPT, _RPT)],
                        out_hbm.at[pl.ds(base, _RPT)])


def _sc_deg(rows3, ones1d, zeros1d):
    mesh = plsc.VectorSubcoreMesh(core_axis_name="c", subcore_axis_name="s")
    return pl.kernel(
        _deg_body,
        out_type=jax.ShapeDtypeStruct((_R * 2 * _NP,), jnp.float32),
        mesh=mesh,
        scratch_types=[
            pltpu.VMEM((_CHUNK,), jnp.float32),
            pltpu.VMEM((_CPT, _CHUNK), jnp.int32),
            pltpu.VMEM_SHARED((_NP,), jnp.float32),
        ],
    )(rows3, ones1d, zeros1d)


def _spmm_body(y_hbm, col_hbm, row_hbm, zeros_hbm, out_hbm,
               colv, rowv, buf, acc, sem):
    cid = lax.axis_index("c")
    sid = lax.axis_index("s")
    wid = sid * 2 + cid
    for r in range(_R):
        pltpu.sync_copy(zeros_hbm, acc.at[pl.ds(sid * _RPT, _RPT)])
        plsc.subcore_barrier()
        pltpu.sync_copy(col_hbm.at[r, wid], colv)
        pltpu.sync_copy(row_hbm.at[r, wid], rowv)

        def chunk(j, carry):
            pltpu.async_copy(y_hbm.at[colv.at[j]], buf, sem).wait()
            pltpu.sync_copy(buf, acc.at[rowv.at[j]], add=True)
            return carry

        lax.fori_loop(0, _CPT, chunk, 0)
        plsc.subcore_barrier()
        pltpu.sync_copy(acc.at[pl.ds(sid * _RPT, _RPT)],
                        out_hbm.at[r, cid, pl.ds(sid * _RPT, _RPT)])


def _sc_spmm(y_flat, cols3, rows3, zeros128):
    mesh = plsc.VectorSubcoreMesh(core_axis_name="c", subcore_axis_name="s")
    return pl.kernel(
        _spmm_body,
        out_type=jax.ShapeDtypeStruct((_R, 2, _NP, _D), jnp.float32),
        mesh=mesh,
        scratch_types=[
            pltpu.VMEM((_CPT, _CHUNK), jnp.int32),
            pltpu.VMEM((_CPT, _CHUNK), jnp.int32),
            pltpu.VMEM((_CHUNK, _D), jnp.float32),
            pltpu.VMEM_SHARED((_NP, _D), jnp.float32),
            pltpu.SemaphoreType.DMA,
        ],
    )(y_flat, cols3, rows3, zeros128)


def _prep_body(x_ref, degp_ref, y1_ref, dinv_ref):
    x = x_ref[...]
    dinvs = []
    for r in range(_R):
        deg = degp_ref[:, 2 * r:2 * r + 1] + degp_ref[:, 2 * r + 1:2 * r + 2]
        dinv = jnp.where(deg > 0, lax.rsqrt(jnp.maximum(deg, 1e-12)), 0.0)
        y1_ref[r] = dinv * x
        dinvs.append(dinv)
    dinv_ref[...] = jnp.concatenate(dinvs, axis=1)


def _tc_prep(X, degp):
    return pl.pallas_call(
        _prep_body,
        grid=(_NP // _BLK,),
        in_specs=[
            pl.BlockSpec((_BLK, _D), lambda i: (i, 0)),
            pl.BlockSpec((_BLK, 2 * _R), lambda i: (i, 0)),
        ],
        out_specs=[
            pl.BlockSpec((_R, _BLK, _D), lambda i: (0, i, 0)),
            pl.BlockSpec((_BLK, _R), lambda i: (i, 0)),
        ],
        out_shape=[
            jax.ShapeDtypeStruct((_R, _NP, _D), jnp.float32),
            jax.ShapeDtypeStruct((_NP, _R), jnp.float32),
        ],
    )(X, degp)


def _mid_body(p_ref, dinv_ref, w1_ref, y2_ref):
    acc = jnp.zeros((_BLK, _D), jnp.float32)
    for r in range(_R):
        z = dinv_ref[:, r:r + 1] * (p_ref[r, 0] + p_ref[r, 1])
        acc += jax.nn.relu(jnp.dot(z, w1_ref[r],
                                   preferred_element_type=jnp.float32))
    h1 = acc * (1.0 / _R)
    for r in range(_R):
        y2_ref[r] = dinv_ref[:, r:r + 1] * h1


def _tc_mid(p1, dinv, W1):
    return pl.pallas_call(
        _mid_body,
        grid=(_NP // _BLK,),
        in_specs=[
            pl.BlockSpec((_R, 2, _BLK, _D), lambda i: (0, 0, i, 0)),
            pl.BlockSpec((_BLK, _R), lambda i: (i, 0)),
            pl.BlockSpec((_R, _D, _D), lambda i: (0, 0, 0)),
        ],
        out_specs=pl.BlockSpec((_R, _BLK, _D), lambda i: (0, i, 0)),
        out_shape=jax.ShapeDtypeStruct((_R, _NP, _D), jnp.float32),
    )(p1, dinv, W1)


def _out_body(p_ref, dinv_ref, w2_ref, q_ref, tau_ref, ow_ref, ob_ref,
              logits_ref, alpha_ref):
    hs = []
    ss = []
    q = q_ref[...]
    for r in range(_R):
        z = dinv_ref[:, r:r + 1] * (p_ref[r, 0] + p_ref[r, 1])
        h = jax.nn.relu(jnp.dot(z, w2_ref[r],
                                preferred_element_type=jnp.float32))
        hs.append(h)
        ss.append(jnp.sum(h * q, axis=1, keepdims=True))
    tau_c = jnp.clip(tau_ref[0, 0], 0.5, 5.0)
    m = jnp.maximum(jnp.maximum(ss[0], ss[1]), ss[2])
    es = [jnp.exp((s - m) / tau_c) for s in ss]
    den = es[0] + es[1] + es[2]
    alphas = [e / den for e in es]
    h2 = (hs[0] + hs[1] + hs[2]) * (1.0 / _R)
    for r in range(_R):
        h2 = h2 + alphas[r] * hs[r]
    logits_ref[...] = (jnp.dot(h2, ow_ref[...],
                               preferred_element_type=jnp.float32)
                       + ob_ref[...])
    alpha_ref[...] = jnp.concatenate(alphas, axis=1)


def _tc_out(p2, dinv, W2, q2d, tau2d, out_W, ob2d):
    return pl.pallas_call(
        _out_body,
        grid=(_NP // _BLK,),
        in_specs=[
            pl.BlockSpec((_R, 2, _BLK, _D), lambda i: (0, 0, i, 0)),
            pl.BlockSpec((_BLK, _R), lambda i: (i, 0)),
            pl.BlockSpec((_R, _D, _D), lambda i: (0, 0, 0)),
            pl.BlockSpec((1, _D), lambda i: (0, 0)),
            pl.BlockSpec((1, 1), lambda i: (0, 0)),
            pl.BlockSpec((_D, _C), lambda i: (0, 0)),
            pl.BlockSpec((1, _C), lambda i: (0, 0)),
        ],
        out_specs=[
            pl.BlockSpec((_BLK, _C), lambda i: (i, 0)),
            pl.BlockSpec((_BLK, _R), lambda i: (i, 0)),
        ],
        out_shape=[
            jax.ShapeDtypeStruct((_NP, _C), jnp.float32),
            jax.ShapeDtypeStruct((_NP, _R), jnp.float32),
        ],
    )(p2, dinv, W2, q2d, tau2d, out_W, ob2d)


@jax.jit
def kernel(X, edge_index_r0, edge_index_r1, edge_index_r2, W1, W2,
           att_q, tau, out_W, out_b):
    edges = jnp.stack([edge_index_r0, edge_index_r1, edge_index_r2])
    rows3 = edges[:, 0, :].reshape(_R, _NW, _CPT, _CHUNK)
    cols3 = (edges[:, 1, :]
             + (jnp.arange(_R, dtype=jnp.int32) * _NP)[:, None]
             ).reshape(_R, _NW, _CPT, _CHUNK)

    ones1d = jnp.ones((_CHUNK,), jnp.float32)
    zeros1d = jnp.zeros((_RPT,), jnp.float32)
    zeros128 = jnp.zeros((_RPT, _D), jnp.float32)
    Xp = jnp.pad(X, ((0, _NP - _N), (0, 0)))

    degp = _sc_deg(rows3, ones1d, zeros1d).reshape(2 * _R, _NP).T
    y1, dinv = _tc_prep(Xp, degp)
    p1 = _sc_spmm(y1.reshape(_R * _NP, _D), cols3, rows3, zeros128)
    y2 = _tc_mid(p1, dinv, W1)
    p2 = _sc_spmm(y2.reshape(_R * _NP, _D), cols3, rows3, zeros128)
    logits, alpha = _tc_out(
        p2, dinv, W2,
        att_q.reshape(1, _D),
        tau.reshape(1, 1),
        out_W,
        out_b.reshape(1, _C),
    )
    return logits[:_N], alpha[:_N]


# speedup vs baseline: 26.0293x; 1.7025x over previous
import functools

import jax
import jax.numpy as jnp
from jax import lax
from jax.experimental import pallas as pl
from jax.experimental.pallas import tpu as pltpu
from jax.experimental.pallas import tpu_sc as plsc

_N = 10000
_E = 320000
_R = 3
_D = 128
_C = 16

_CHUNK = 125
_NW = 32
_CPT = _E // (_CHUNK * _NW)
_HCPT = _CPT // 2
_NP = 10240
_RPT = _NP // 16

_BLK = 1024


def _deg_body(row_hbm, ones_hbm, zeros_hbm, out_hbm, onesv, rowv, acc):
    cid = lax.axis_index("c")
    sid = lax.axis_index("s")
    wid = sid * 2 + cid
    pltpu.sync_copy(ones_hbm, onesv)
    for r in range(_R):
        pltpu.sync_copy(zeros_hbm, acc.at[pl.ds(sid * _RPT, _RPT)])
        plsc.subcore_barrier()
        pltpu.sync_copy(row_hbm.at[r, wid], rowv)

        def chunk(j, carry):
            pltpu.sync_copy(onesv, acc.at[rowv.at[j]], add=True)
            return carry

        lax.fori_loop(0, _CPT, chunk, 0)
        plsc.subcore_barrier()
        base = pl.multiple_of((r * 2 + cid) * _NP + sid * _RPT, 128)
        pltpu.sync_copy(acc.at[pl.ds(sid * _RPT, _RPT)],
                        out_hbm.at[pl.ds(base, _RPT)])


def _sc_deg(rows3, ones1d, zeros1d):
    mesh = plsc.VectorSubcoreMesh(core_axis_name="c", subcore_axis_name="s")
    return pl.kernel(
        _deg_body,
        out_type=jax.ShapeDtypeStruct((_R * 2 * _NP,), jnp.float32),
        mesh=mesh,
        scratch_types=[
            pltpu.VMEM((_CHUNK,), jnp.float32),
            pltpu.VMEM((_CPT, _CHUNK), jnp.int32),
            pltpu.VMEM_SHARED((_NP,), jnp.float32),
        ],
    )(rows3, ones1d, zeros1d)


def _spmm_body(y_hbm, col_hbm, row_hbm, zeros_hbm, out_hbm,
               colv, rowv, buf_a, buf_b, acc, sem_a, sem_b):
    cid = lax.axis_index("c")
    sid = lax.axis_index("s")
    wid = sid * 2 + cid
    for r in range(_R):
        pltpu.sync_copy(zeros_hbm, acc.at[pl.ds(sid * _RPT, _RPT)])
        plsc.subcore_barrier()
        for h in range(2):
            pltpu.sync_copy(col_hbm.at[r, wid, pl.ds(h * _HCPT, _HCPT)],
                            colv)
            pltpu.sync_copy(row_hbm.at[r, wid, pl.ds(h * _HCPT, _HCPT)],
                            rowv)

            pltpu.async_copy(y_hbm.at[colv.at[0]], buf_a, sem_a)

            def pair(k, carry):
                j0 = 2 * k
                pltpu.async_copy(y_hbm.at[colv.at[j0 + 1]], buf_b, sem_b)
                pltpu.make_async_copy(
                    y_hbm.at[colv.at[j0]], buf_a, sem_a).wait()
                pltpu.sync_copy(buf_a, acc.at[rowv.at[j0]], add=True)
                pltpu.async_copy(y_hbm.at[colv.at[j0 + 2]], buf_a, sem_a)
                pltpu.make_async_copy(
                    y_hbm.at[colv.at[j0 + 1]], buf_b, sem_b).wait()
                pltpu.sync_copy(buf_b, acc.at[rowv.at[j0 + 1]], add=True)
                return carry

            lax.fori_loop(0, _HCPT // 2 - 1, pair, 0)
            jlast = _HCPT - 2
            pltpu.async_copy(y_hbm.at[colv.at[jlast + 1]], buf_b, sem_b)
            pltpu.make_async_copy(
                y_hbm.at[colv.at[jlast]], buf_a, sem_a).wait()
            pltpu.sync_copy(buf_a, acc.at[rowv.at[jlast]], add=True)
            pltpu.make_async_copy(
                y_hbm.at[colv.at[jlast + 1]], buf_b, sem_b).wait()
            pltpu.sync_copy(buf_b, acc.at[rowv.at[jlast + 1]], add=True)

        plsc.subcore_barrier()
        pltpu.sync_copy(acc.at[pl.ds(sid * _RPT, _RPT)],
                        out_hbm.at[r, cid, pl.ds(sid * _RPT, _RPT)])


def _sc_spmm(y_flat, cols3, rows3, zeros128):
    mesh = plsc.VectorSubcoreMesh(core_axis_name="c", subcore_axis_name="s")
    return pl.kernel(
        _spmm_body,
        out_type=jax.ShapeDtypeStruct((_R, 2, _NP, _D), jnp.float32),
        mesh=mesh,
        scratch_types=[
            pltpu.VMEM((_HCPT, _CHUNK), jnp.int32),
            pltpu.VMEM((_HCPT, _CHUNK), jnp.int32),
            pltpu.VMEM((_CHUNK, _D), jnp.float32),
            pltpu.VMEM((_CHUNK, _D), jnp.float32),
            pltpu.VMEM_SHARED((_NP, _D), jnp.float32),
            pltpu.SemaphoreType.DMA,
            pltpu.SemaphoreType.DMA,
        ],
    )(y_flat, cols3, rows3, zeros128)


def _prep_body(x_ref, degp_ref, y1_ref, dinv_ref):
    x = x_ref[...]
    dinvs = []
    for r in range(_R):
        deg = degp_ref[:, 2 * r:2 * r + 1] + degp_ref[:, 2 * r + 1:2 * r + 2]
        dinv = jnp.where(deg > 0, lax.rsqrt(jnp.maximum(deg, 1e-12)), 0.0)
        y1_ref[r] = dinv * x
        dinvs.append(dinv)
    dinv_ref[...] = jnp.concatenate(dinvs, axis=1)


def _tc_prep(X, degp):
    return pl.pallas_call(
        _prep_body,
        grid=(_NP // _BLK,),
        in_specs=[
            pl.BlockSpec((_BLK, _D), lambda i: (i, 0)),
            pl.BlockSpec((_BLK, 2 * _R), lambda i: (i, 0)),
        ],
        out_specs=[
            pl.BlockSpec((_R, _BLK, _D), lambda i: (0, i, 0)),
            pl.BlockSpec((_BLK, _R), lambda i: (i, 0)),
        ],
        out_shape=[
            jax.ShapeDtypeStruct((_R, _NP, _D), jnp.float32),
            jax.ShapeDtypeStruct((_NP, _R), jnp.float32),
        ],
    )(X, degp)


def _mid_body(p_ref, dinv_ref, w1_ref, y2_ref):
    acc = jnp.zeros((_BLK, _D), jnp.float32)
    for r in range(_R):
        z = dinv_ref[:, r:r + 1] * (p_ref[r, 0] + p_ref[r, 1])
        acc += jax.nn.relu(jnp.dot(z, w1_ref[r],
                                   preferred_element_type=jnp.float32))
    h1 = acc * (1.0 / _R)
    for r in range(_R):
        y2_ref[r] = dinv_ref[:, r:r + 1] * h1


def _tc_mid(p1, dinv, W1):
    return pl.pallas_call(
        _mid_body,
        grid=(_NP // _BLK,),
        in_specs=[
            pl.BlockSpec((_R, 2, _BLK, _D), lambda i: (0, 0, i, 0)),
            pl.BlockSpec((_BLK, _R), lambda i: (i, 0)),
            pl.BlockSpec((_R, _D, _D), lambda i: (0, 0, 0)),
        ],
        out_specs=pl.BlockSpec((_R, _BLK, _D), lambda i: (0, i, 0)),
        out_shape=jax.ShapeDtypeStruct((_R, _NP, _D), jnp.float32),
    )(p1, dinv, W1)


def _out_body(p_ref, dinv_ref, w2_ref, q_ref, tau_ref, ow_ref, ob_ref,
              logits_ref, alpha_ref):
    hs = []
    ss = []
    q = q_ref[...]
    for r in range(_R):
        z = dinv_ref[:, r:r + 1] * (p_ref[r, 0] + p_ref[r, 1])
        h = jax.nn.relu(jnp.dot(z, w2_ref[r],
                                preferred_element_type=jnp.float32))
        hs.append(h)
        ss.append(jnp.sum(h * q, axis=1, keepdims=True))
    tau_c = jnp.clip(tau_ref[0, 0], 0.5, 5.0)
    m = jnp.maximum(jnp.maximum(ss[0], ss[1]), ss[2])
    es = [jnp.exp((s - m) / tau_c) for s in ss]
    den = es[0] + es[1] + es[2]
    alphas = [e / den for e in es]
    h2 = (hs[0] + hs[1] + hs[2]) * (1.0 / _R)
    for r in range(_R):
        h2 = h2 + alphas[r] * hs[r]
    logits_ref[...] = (jnp.dot(h2, ow_ref[...],
                               preferred_element_type=jnp.float32)
                       + ob_ref[...])
    alpha_ref[...] = jnp.concatenate(alphas, axis=1)


def _tc_out(p2, dinv, W2, q2d, tau2d, out_W, ob2d):
    return pl.pallas_call(
        _out_body,
        grid=(_NP // _BLK,),
        in_specs=[
            pl.BlockSpec((_R, 2, _BLK, _D), lambda i: (0, 0, i, 0)),
            pl.BlockSpec((_BLK, _R), lambda i: (i, 0)),
            pl.BlockSpec((_R, _D, _D), lambda i: (0, 0, 0)),
            pl.BlockSpec((1, _D), lambda i: (0, 0)),
            pl.BlockSpec((1, 1), lambda i: (0, 0)),
            pl.BlockSpec((_D, _C), lambda i: (0, 0)),
            pl.BlockSpec((1, _C), lambda i: (0, 0)),
        ],
        out_specs=[
            pl.BlockSpec((_BLK, _C), lambda i: (i, 0)),
            pl.BlockSpec((_BLK, _R), lambda i: (i, 0)),
        ],
        out_shape=[
            jax.ShapeDtypeStruct((_NP, _C), jnp.float32),
            jax.ShapeDtypeStruct((_NP, _R), jnp.float32),
        ],
    )(p2, dinv, W2, q2d, tau2d, out_W, ob2d)


@jax.jit
def kernel(X, edge_index_r0, edge_index_r1, edge_index_r2, W1, W2,
           att_q, tau, out_W, out_b):
    edges = jnp.stack([edge_index_r0, edge_index_r1, edge_index_r2])
    rows3 = edges[:, 0, :].reshape(_R, _NW, _CPT, _CHUNK)
    cols3 = (edges[:, 1, :]
             + (jnp.arange(_R, dtype=jnp.int32) * _NP)[:, None]
             ).reshape(_R, _NW, _CPT, _CHUNK)

    ones1d = jnp.ones((_CHUNK,), jnp.float32)
    zeros1d = jnp.zeros((_RPT,), jnp.float32)
    zeros128 = jnp.zeros((_RPT, _D), jnp.float32)
    Xp = jnp.pad(X, ((0, _NP - _N), (0, 0)))

    degp = _sc_deg(rows3, ones1d, zeros1d).reshape(2 * _R, _NP).T
    y1, dinv = _tc_prep(Xp, degp)
    p1 = _sc_spmm(y1.reshape(_R * _NP, _D), cols3, rows3, zeros128)
    y2 = _tc_mid(p1, dinv, W1)
    p2 = _sc_spmm(y2.reshape(_R * _NP, _D), cols3, rows3, zeros128)
    logits, alpha = _tc_out(
        p2, dinv, W2,
        att_q.reshape(1, _D),
        tau.reshape(1, 1),
        out_W,
        out_b.reshape(1, _C),
    )
    return logits[:_N], alpha[:_N]


# speedup vs baseline: 26.6378x; 1.0234x over previous
import functools

import jax
import jax.numpy as jnp
from jax import lax
from jax.experimental import pallas as pl
from jax.experimental.pallas import tpu as pltpu
from jax.experimental.pallas import tpu_sc as plsc

_N = 10000
_E = 320000
_R = 3
_D = 128
_C = 16

_CHUNK = 125
_NW = 32
_CPT = _E // (_CHUNK * _NW)
_HCPT = _CPT // 2
_NP = 10240
_RPT = _NP // 16

_BLK = 1024


def _deg_body(row_hbm, ones_hbm, zeros_hbm, out_hbm, onesv, rowv, acc,
              sem_s):
    cid = lax.axis_index("c")
    sid = lax.axis_index("s")
    wid = sid * 2 + cid
    pltpu.sync_copy(ones_hbm, onesv)
    for r in range(_R):
        pltpu.sync_copy(zeros_hbm, acc.at[pl.ds(sid * _RPT, _RPT)])
        plsc.subcore_barrier()
        pltpu.sync_copy(row_hbm.at[r, wid], rowv)

        for jj in range(4):
            pltpu.async_copy(onesv, acc.at[rowv.at[jj]], sem_s, add=True)

        def chunk(j, carry):
            pltpu.make_async_copy(onesv, acc.at[rowv.at[j]], sem_s).wait()
            pltpu.async_copy(onesv, acc.at[rowv.at[j + 4]], sem_s, add=True)
            return carry

        lax.fori_loop(0, _CPT - 4, chunk, 0)
        for jj in range(_CPT - 4, _CPT):
            pltpu.make_async_copy(onesv, acc.at[rowv.at[jj]], sem_s).wait()
        plsc.subcore_barrier()
        base = pl.multiple_of((r * 2 + cid) * _NP + sid * _RPT, 128)
        pltpu.sync_copy(acc.at[pl.ds(sid * _RPT, _RPT)],
                        out_hbm.at[pl.ds(base, _RPT)])


def _sc_deg(rows3, ones1d, zeros1d):
    mesh = plsc.VectorSubcoreMesh(core_axis_name="c", subcore_axis_name="s")
    return pl.kernel(
        _deg_body,
        out_type=jax.ShapeDtypeStruct((_R * 2 * _NP,), jnp.float32),
        mesh=mesh,
        scratch_types=[
            pltpu.VMEM((_CHUNK,), jnp.float32),
            pltpu.VMEM((_CPT, _CHUNK), jnp.int32),
            pltpu.VMEM_SHARED((_NP,), jnp.float32),
            pltpu.SemaphoreType.DMA,
        ],
    )(rows3, ones1d, zeros1d)


def _spmm_body(y_hbm, idx_hbm, zeros_hbm, out_hbm,
               idxv, buf_a, buf_b, acc, sem_a, sem_b):
    cid = lax.axis_index("c")
    sid = lax.axis_index("s")
    wid = sid * 2 + cid
    for r in range(_R):
        pltpu.sync_copy(zeros_hbm, acc.at[pl.ds(sid * _RPT, _RPT)])
        plsc.subcore_barrier()
        for h in range(2):
            pltpu.sync_copy(idx_hbm.at[r, wid, h], idxv)
            colv = idxv.at[0]
            rowv = idxv.at[1]

            pltpu.async_copy(y_hbm.at[colv.at[0]], buf_a, sem_a)

            def pair(k, carry):
                j0 = 2 * k
                pltpu.async_copy(y_hbm.at[colv.at[j0 + 1]], buf_b, sem_b)
                pltpu.make_async_copy(
                    y_hbm.at[colv.at[j0]], buf_a, sem_a).wait()
                pltpu.sync_copy(buf_a, acc.at[rowv.at[j0]], add=True)
                pltpu.async_copy(y_hbm.at[colv.at[j0 + 2]], buf_a, sem_a)
                pltpu.make_async_copy(
                    y_hbm.at[colv.at[j0 + 1]], buf_b, sem_b).wait()
                pltpu.sync_copy(buf_b, acc.at[rowv.at[j0 + 1]], add=True)
                return carry

            lax.fori_loop(0, _HCPT // 2 - 1, pair, 0)
            jlast = _HCPT - 2
            pltpu.async_copy(y_hbm.at[colv.at[jlast + 1]], buf_b, sem_b)
            pltpu.make_async_copy(
                y_hbm.at[colv.at[jlast]], buf_a, sem_a).wait()
            pltpu.sync_copy(buf_a, acc.at[rowv.at[jlast]], add=True)
            pltpu.make_async_copy(
                y_hbm.at[colv.at[jlast + 1]], buf_b, sem_b).wait()
            pltpu.sync_copy(buf_b, acc.at[rowv.at[jlast + 1]], add=True)

        plsc.subcore_barrier()
        pltpu.sync_copy(acc.at[pl.ds(sid * _RPT, _RPT)],
                        out_hbm.at[r, cid, pl.ds(sid * _RPT, _RPT)])


def _sc_spmm(y_flat, idx3, zeros128):
    mesh = plsc.VectorSubcoreMesh(core_axis_name="c", subcore_axis_name="s")
    return pl.kernel(
        _spmm_body,
        out_type=jax.ShapeDtypeStruct((_R, 2, _NP, _D), jnp.float32),
        mesh=mesh,
        scratch_types=[
            pltpu.VMEM((2, _HCPT, _CHUNK), jnp.int32),
            pltpu.VMEM((_CHUNK, _D), jnp.float32),
            pltpu.VMEM((_CHUNK, _D), jnp.float32),
            pltpu.VMEM_SHARED((_NP, _D), jnp.float32),
            pltpu.SemaphoreType.DMA,
            pltpu.SemaphoreType.DMA,
        ],
    )(y_flat, idx3, zeros128)


def _prep_body(x_ref, degp_ref, y1_ref, dinv_ref):
    x = x_ref[...]
    dinvs = []
    for r in range(_R):
        deg = degp_ref[:, 2 * r:2 * r + 1] + degp_ref[:, 2 * r + 1:2 * r + 2]
        dinv = jnp.where(deg > 0, lax.rsqrt(jnp.maximum(deg, 1e-12)), 0.0)
        y1_ref[r] = dinv * x
        dinvs.append(dinv)
    dinv_ref[...] = jnp.concatenate(dinvs, axis=1)


def _tc_prep(X, degp):
    return pl.pallas_call(
        _prep_body,
        grid=(_NP // _BLK,),
        in_specs=[
            pl.BlockSpec((_BLK, _D), lambda i: (i, 0)),
            pl.BlockSpec((_BLK, 2 * _R), lambda i: (i, 0)),
        ],
        out_specs=[
            pl.BlockSpec((_R, _BLK, _D), lambda i: (0, i, 0)),
            pl.BlockSpec((_BLK, _R), lambda i: (i, 0)),
        ],
        out_shape=[
            jax.ShapeDtypeStruct((_R, _NP, _D), jnp.float32),
            jax.ShapeDtypeStruct((_NP, _R), jnp.float32),
        ],
    )(X, degp)


def _mid_body(p_ref, dinv_ref, w1_ref, y2_ref):
    acc = jnp.zeros((_BLK, _D), jnp.float32)
    for r in range(_R):
        z = dinv_ref[:, r:r + 1] * (p_ref[r, 0] + p_ref[r, 1])
        acc += jax.nn.relu(jnp.dot(z, w1_ref[r],
                                   preferred_element_type=jnp.float32))
    h1 = acc * (1.0 / _R)
    for r in range(_R):
        y2_ref[r] = dinv_ref[:, r:r + 1] * h1


def _tc_mid(p1, dinv, W1):
    return pl.pallas_call(
        _mid_body,
        grid=(_NP // _BLK,),
        in_specs=[
            pl.BlockSpec((_R, 2, _BLK, _D), lambda i: (0, 0, i, 0)),
            pl.BlockSpec((_BLK, _R), lambda i: (i, 0)),
            pl.BlockSpec((_R, _D, _D), lambda i: (0, 0, 0)),
        ],
        out_specs=pl.BlockSpec((_R, _BLK, _D), lambda i: (0, i, 0)),
        out_shape=jax.ShapeDtypeStruct((_R, _NP, _D), jnp.float32),
    )(p1, dinv, W1)


def _out_body(p_ref, dinv_ref, w2_ref, q_ref, tau_ref, ow_ref, ob_ref,
              logits_ref, alpha_ref):
    hs = []
    ss = []
    q = q_ref[...]
    for r in range(_R):
        z = dinv_ref[:, r:r + 1] * (p_ref[r, 0] + p_ref[r, 1])
        h = jax.nn.relu(jnp.dot(z, w2_ref[r],
                                preferred_element_type=jnp.float32))
        hs.append(h)
        ss.append(jnp.sum(h * q, axis=1, keepdims=True))
    tau_c = jnp.clip(tau_ref[0, 0], 0.5, 5.0)
    m = jnp.maximum(jnp.maximum(ss[0], ss[1]), ss[2])
    es = [jnp.exp((s - m) / tau_c) for s in ss]
    den = es[0] + es[1] + es[2]
    alphas = [e / den for e in es]
    h2 = (hs[0] + hs[1] + hs[2]) * (1.0 / _R)
    for r in range(_R):
        h2 = h2 + alphas[r] * hs[r]
    logits_ref[...] = (jnp.dot(h2, ow_ref[...],
                               preferred_element_type=jnp.float32)
                       + ob_ref[...])
    alpha_ref[...] = jnp.concatenate(alphas, axis=1)


def _tc_out(p2, dinv, W2, q2d, tau2d, out_W, ob2d):
    return pl.pallas_call(
        _out_body,
        grid=(_NP // _BLK,),
        in_specs=[
            pl.BlockSpec((_R, 2, _BLK, _D), lambda i: (0, 0, i, 0)),
            pl.BlockSpec((_BLK, _R), lambda i: (i, 0)),
            pl.BlockSpec((_R, _D, _D), lambda i: (0, 0, 0)),
            pl.BlockSpec((1, _D), lambda i: (0, 0)),
            pl.BlockSpec((1, 1), lambda i: (0, 0)),
            pl.BlockSpec((_D, _C), lambda i: (0, 0)),
            pl.BlockSpec((1, _C), lambda i: (0, 0)),
        ],
        out_specs=[
            pl.BlockSpec((_BLK, _C), lambda i: (i, 0)),
            pl.BlockSpec((_BLK, _R), lambda i: (i, 0)),
        ],
        out_shape=[
            jax.ShapeDtypeStruct((_NP, _C), jnp.float32),
            jax.ShapeDtypeStruct((_NP, _R), jnp.float32),
        ],
    )(p2, dinv, W2, q2d, tau2d, out_W, ob2d)


@jax.jit
def kernel(X, edge_index_r0, edge_index_r1, edge_index_r2, W1, W2,
           att_q, tau, out_W, out_b):
    edges = jnp.stack([edge_index_r0, edge_index_r1, edge_index_r2])
    rows3 = edges[:, 0, :].reshape(_R, _NW, _CPT, _CHUNK)
    cols3 = (edges[:, 1, :]
             + (jnp.arange(_R, dtype=jnp.int32) * _NP)[:, None]
             ).reshape(_R, _NW, _CPT, _CHUNK)
    idx3 = jnp.stack(
        [cols3.reshape(_R, _NW, 2, _HCPT, _CHUNK),
         rows3.reshape(_R, _NW, 2, _HCPT, _CHUNK)], axis=3)

    ones1d = jnp.ones((_CHUNK,), jnp.float32)
    zeros1d = jnp.zeros((_RPT,), jnp.float32)
    zeros128 = jnp.zeros((_RPT, _D), jnp.float32)
    Xp = jnp.pad(X, ((0, _NP - _N), (0, 0)))

    degp = _sc_deg(rows3, ones1d, zeros1d).reshape(2 * _R, _NP).T
    y1, dinv = _tc_prep(Xp, degp)
    p1 = _sc_spmm(y1.reshape(_R * _NP, _D), idx3, zeros128)
    y2 = _tc_mid(p1, dinv, W1)
    p2 = _sc_spmm(y2.reshape(_R * _NP, _D), idx3, zeros128)
    logits, alpha = _tc_out(
        p2, dinv, W2,
        att_q.reshape(1, _D),
        tau.reshape(1, 1),
        out_W,
        out_b.reshape(1, _C),
    )
    return logits[:_N], alpha[:_N]


# speedup vs baseline: 26.7575x; 1.0045x over previous
import functools

import jax
import jax.numpy as jnp
from jax import lax
from jax.experimental import pallas as pl
from jax.experimental.pallas import tpu as pltpu
from jax.experimental.pallas import tpu_sc as plsc

_N = 10000
_E = 320000
_R = 3
_D = 128
_C = 16

_CHUNK = 125
_NW = 32
_CPT = _E // (_CHUNK * _NW)
_HCPT = _CPT // 2
_NP = 10240
_RPT = _NP // 16

_BLK = 1024


def _deg_body(e0_hbm, e1_hbm, e2_hbm, ones_hbm, zeros_hbm, out_hbm,
              onesv, rowv, acc, sem_s):
    cid = lax.axis_index("c")
    sid = lax.axis_index("s")
    wid = sid * 2 + cid
    pltpu.sync_copy(ones_hbm, onesv)
    edges = [e0_hbm, e1_hbm, e2_hbm]
    for r in range(_R):
        pltpu.sync_copy(zeros_hbm, acc.at[pl.ds(sid * _RPT, _RPT)])
        plsc.subcore_barrier()
        for h in range(2):
            pltpu.sync_copy(edges[r].at[0, wid, h],
                            rowv.at[pl.ds(h * _HCPT, _HCPT)])

        for jj in range(4):
            pltpu.async_copy(onesv, acc.at[rowv.at[jj]], sem_s, add=True)

        def chunk(j, carry):
            pltpu.make_async_copy(onesv, acc.at[rowv.at[j]], sem_s).wait()
            pltpu.async_copy(onesv, acc.at[rowv.at[j + 4]], sem_s, add=True)
            return carry

        lax.fori_loop(0, _CPT - 4, chunk, 0)
        for jj in range(_CPT - 4, _CPT):
            pltpu.make_async_copy(onesv, acc.at[rowv.at[jj]], sem_s).wait()
        plsc.subcore_barrier()
        base = pl.multiple_of((r * 2 + cid) * _NP + sid * _RPT, 128)
        pltpu.sync_copy(acc.at[pl.ds(sid * _RPT, _RPT)],
                        out_hbm.at[pl.ds(base, _RPT)])


def _sc_deg(e0, e1, e2, ones1d, zeros1d):
    mesh = plsc.VectorSubcoreMesh(core_axis_name="c", subcore_axis_name="s")
    return pl.kernel(
        _deg_body,
        out_type=jax.ShapeDtypeStruct((_R * 2 * _NP,), jnp.float32),
        mesh=mesh,
        scratch_types=[
            pltpu.VMEM((_CHUNK,), jnp.float32),
            pltpu.VMEM((_CPT, _CHUNK), jnp.int32),
            pltpu.VMEM_SHARED((_NP,), jnp.float32),
            pltpu.SemaphoreType.DMA,
        ],
    )(e0, e1, e2, ones1d, zeros1d)


def _spmm_body(y0_hbm, y1_hbm, y2_hbm, e0_hbm, e1_hbm, e2_hbm,
               zeros_hbm, out_hbm,
               idxv, buf_a, buf_b, acc, sem_a, sem_b):
    cid = lax.axis_index("c")
    sid = lax.axis_index("s")
    wid = sid * 2 + cid
    ys = [y0_hbm, y1_hbm, y2_hbm]
    edges = [e0_hbm, e1_hbm, e2_hbm]
    for r in range(_R):
        y_hbm = ys[r]
        pltpu.sync_copy(zeros_hbm, acc.at[pl.ds(sid * _RPT, _RPT)])
        plsc.subcore_barrier()
        for h in range(2):
            pltpu.sync_copy(edges[r].at[0, wid, h], idxv.at[1])
            pltpu.sync_copy(edges[r].at[1, wid, h], idxv.at[0])
            colv = idxv.at[0]
            rowv = idxv.at[1]

            pltpu.async_copy(y_hbm.at[colv.at[0]], buf_a, sem_a)

            def pair(k, carry):
                j0 = 2 * k
                pltpu.async_copy(y_hbm.at[colv.at[j0 + 1]], buf_b, sem_b)
                pltpu.make_async_copy(
                    y_hbm.at[colv.at[j0]], buf_a, sem_a).wait()
                pltpu.sync_copy(buf_a, acc.at[rowv.at[j0]], add=True)
                pltpu.async_copy(y_hbm.at[colv.at[j0 + 2]], buf_a, sem_a)
                pltpu.make_async_copy(
                    y_hbm.at[colv.at[j0 + 1]], buf_b, sem_b).wait()
                pltpu.sync_copy(buf_b, acc.at[rowv.at[j0 + 1]], add=True)
                return carry

            lax.fori_loop(0, _HCPT // 2 - 1, pair, 0)
            jlast = _HCPT - 2
            pltpu.async_copy(y_hbm.at[colv.at[jlast + 1]], buf_b, sem_b)
            pltpu.make_async_copy(
                y_hbm.at[colv.at[jlast]], buf_a, sem_a).wait()
            pltpu.sync_copy(buf_a, acc.at[rowv.at[jlast]], add=True)
            pltpu.make_async_copy(
                y_hbm.at[colv.at[jlast + 1]], buf_b, sem_b).wait()
            pltpu.sync_copy(buf_b, acc.at[rowv.at[jlast + 1]], add=True)

        plsc.subcore_barrier()
        pltpu.sync_copy(acc.at[pl.ds(sid * _RPT, _RPT)],
                        out_hbm.at[r, cid, pl.ds(sid * _RPT, _RPT)])


def _sc_spmm(y3, e0, e1, e2, zeros128):
    mesh = plsc.VectorSubcoreMesh(core_axis_name="c", subcore_axis_name="s")
    return pl.kernel(
        _spmm_body,
        out_type=jax.ShapeDtypeStruct((_R, 2, _NP, _D), jnp.float32),
        mesh=mesh,
        scratch_types=[
            pltpu.VMEM((2, _HCPT, _CHUNK), jnp.int32),
            pltpu.VMEM((_CHUNK, _D), jnp.float32),
            pltpu.VMEM((_CHUNK, _D), jnp.float32),
            pltpu.VMEM_SHARED((_NP, _D), jnp.float32),
            pltpu.SemaphoreType.DMA,
            pltpu.SemaphoreType.DMA,
        ],
    )(y3[0], y3[1], y3[2], e0, e1, e2, zeros128)


def _prep_body(x_ref, degp_ref, y1_ref, dinv_ref):
    x = x_ref[...]
    dinvs = []
    for r in range(_R):
        deg = degp_ref[:, 2 * r:2 * r + 1] + degp_ref[:, 2 * r + 1:2 * r + 2]
        dinv = jnp.where(deg > 0, lax.rsqrt(jnp.maximum(deg, 1e-12)), 0.0)
        y1_ref[r] = dinv * x
        dinvs.append(dinv)
    dinv_ref[...] = jnp.concatenate(dinvs, axis=1)


def _tc_prep(X, degp):
    return pl.pallas_call(
        _prep_body,
        grid=(_NP // _BLK,),
        in_specs=[
            pl.BlockSpec((_BLK, _D), lambda i: (i, 0)),
            pl.BlockSpec((_BLK, 2 * _R), lambda i: (i, 0)),
        ],
        out_specs=[
            pl.BlockSpec((_R, _BLK, _D), lambda i: (0, i, 0)),
            pl.BlockSpec((_BLK, _R), lambda i: (i, 0)),
        ],
        out_shape=[
            jax.ShapeDtypeStruct((_R, _NP, _D), jnp.float32),
            jax.ShapeDtypeStruct((_NP, _R), jnp.float32),
        ],
    )(X, degp)


def _mid_body(p_ref, dinv_ref, w1_ref, y2_ref):
    acc = jnp.zeros((_BLK, _D), jnp.float32)
    for r in range(_R):
        z = dinv_ref[:, r:r + 1] * (p_ref[r, 0] + p_ref[r, 1])
        acc += jax.nn.relu(jnp.dot(z, w1_ref[r],
                                   preferred_element_type=jnp.float32))
    h1 = acc * (1.0 / _R)
    for r in range(_R):
        y2_ref[r] = dinv_ref[:, r:r + 1] * h1


def _tc_mid(p1, dinv, W1):
    return pl.pallas_call(
        _mid_body,
        grid=(_NP // _BLK,),
        in_specs=[
            pl.BlockSpec((_R, 2, _BLK, _D), lambda i: (0, 0, i, 0)),
            pl.BlockSpec((_BLK, _R), lambda i: (i, 0)),
            pl.BlockSpec((_R, _D, _D), lambda i: (0, 0, 0)),
        ],
        out_specs=pl.BlockSpec((_R, _BLK, _D), lambda i: (0, i, 0)),
        out_shape=jax.ShapeDtypeStruct((_R, _NP, _D), jnp.float32),
    )(p1, dinv, W1)


def _out_body(p_ref, dinv_ref, w2_ref, q_ref, tau_ref, ow_ref, ob_ref,
              logits_ref, alpha_ref):
    hs = []
    ss = []
    q = q_ref[...]
    for r in range(_R):
        z = dinv_ref[:, r:r + 1] * (p_ref[r, 0] + p_ref[r, 1])
        h = jax.nn.relu(jnp.dot(z, w2_ref[r],
                                preferred_element_type=jnp.float32))
        hs.append(h)
        ss.append(jnp.sum(h * q, axis=1, keepdims=True))
    tau_c = jnp.clip(tau_ref[0, 0], 0.5, 5.0)
    m = jnp.maximum(jnp.maximum(ss[0], ss[1]), ss[2])
    es = [jnp.exp((s - m) / tau_c) for s in ss]
    den = es[0] + es[1] + es[2]
    alphas = [e / den for e in es]
    h2 = (hs[0] + hs[1] + hs[2]) * (1.0 / _R)
    for r in range(_R):
        h2 = h2 + alphas[r] * hs[r]
    logits_ref[...] = (jnp.dot(h2, ow_ref[...],
                               preferred_element_type=jnp.float32)
                       + ob_ref[...])
    alpha_ref[...] = jnp.concatenate(alphas, axis=1)


def _tc_out(p2, dinv, W2, q2d, tau2d, out_W, ob2d):
    return pl.pallas_call(
        _out_body,
        grid=(_NP // _BLK,),
        in_specs=[
            pl.BlockSpec((_R, 2, _BLK, _D), lambda i: (0, 0, i, 0)),
            pl.BlockSpec((_BLK, _R), lambda i: (i, 0)),
            pl.BlockSpec((_R, _D, _D), lambda i: (0, 0, 0)),
            pl.BlockSpec((1, _D), lambda i: (0, 0)),
            pl.BlockSpec((1, 1), lambda i: (0, 0)),
            pl.BlockSpec((_D, _C), lambda i: (0, 0)),
            pl.BlockSpec((1, _C), lambda i: (0, 0)),
        ],
        out_specs=[
            pl.BlockSpec((_BLK, _C), lambda i: (i, 0)),
            pl.BlockSpec((_BLK, _R), lambda i: (i, 0)),
        ],
        out_shape=[
            jax.ShapeDtypeStruct((_NP, _C), jnp.float32),
            jax.ShapeDtypeStruct((_NP, _R), jnp.float32),
        ],
    )(p2, dinv, W2, q2d, tau2d, out_W, ob2d)


@jax.jit
def kernel(X, edge_index_r0, edge_index_r1, edge_index_r2, W1, W2,
           att_q, tau, out_W, out_b):
    e0 = edge_index_r0.reshape(2, _NW, 2, _HCPT, _CHUNK)
    e1 = edge_index_r1.reshape(2, _NW, 2, _HCPT, _CHUNK)
    e2 = edge_index_r2.reshape(2, _NW, 2, _HCPT, _CHUNK)

    ones1d = jnp.ones((_CHUNK,), jnp.float32)
    zeros1d = jnp.zeros((_RPT,), jnp.float32)
    zeros128 = jnp.zeros((_RPT, _D), jnp.float32)
    Xp = jnp.pad(X, ((0, _NP - _N), (0, 0)))

    degp = _sc_deg(e0, e1, e2, ones1d, zeros1d).reshape(2 * _R, _NP).T
    y1, dinv = _tc_prep(Xp, degp)
    p1 = _sc_spmm(y1, e0, e1, e2, zeros128)
    y2 = _tc_mid(p1, dinv, W1)
    p2 = _sc_spmm(y2, e0, e1, e2, zeros128)
    logits, alpha = _tc_out(
        p2, dinv, W2,
        att_q.reshape(1, _D),
        tau.reshape(1, 1),
        out_W,
        out_b.reshape(1, _C),
    )
    return logits[:_N], alpha[:_N]


# speedup vs baseline: 27.6293x; 1.0326x over previous
import functools

import jax
import jax.numpy as jnp
from jax import lax
from jax.experimental import pallas as pl
from jax.experimental.pallas import tpu as pltpu
from jax.experimental.pallas import tpu_sc as plsc

_N = 10000
_E = 320000
_R = 3
_D = 128
_C = 16

_CHUNK = 125
_NW = 32
_CPT = _E // (_CHUNK * _NW)
_HCPT = _CPT // 2
_NP = 10240
_RPT = _NP // 16

_BLK = 1000


def _deg_body(e0_hbm, e1_hbm, e2_hbm, ones_hbm, zeros_hbm, out_hbm,
              onesv, rowv, acc, sem_s):
    cid = lax.axis_index("c")
    sid = lax.axis_index("s")
    wid = sid * 2 + cid
    pltpu.sync_copy(ones_hbm, onesv)
    edges = [e0_hbm, e1_hbm, e2_hbm]
    for r in range(_R):
        pltpu.sync_copy(zeros_hbm, acc.at[pl.ds(sid * _RPT, _RPT)])
        plsc.subcore_barrier()
        for h in range(2):
            pltpu.sync_copy(edges[r].at[0, wid, h],
                            rowv.at[pl.ds(h * _HCPT, _HCPT)])

        for jj in range(4):
            pltpu.async_copy(onesv, acc.at[rowv.at[jj]], sem_s, add=True)

        def chunk(j, carry):
            pltpu.make_async_copy(onesv, acc.at[rowv.at[j]], sem_s).wait()
            pltpu.async_copy(onesv, acc.at[rowv.at[j + 4]], sem_s, add=True)
            return carry

        lax.fori_loop(0, _CPT - 4, chunk, 0)
        for jj in range(_CPT - 4, _CPT):
            pltpu.make_async_copy(onesv, acc.at[rowv.at[jj]], sem_s).wait()
        plsc.subcore_barrier()
        base = pl.multiple_of((r * 2 + cid) * _NP + sid * _RPT, 128)
        pltpu.sync_copy(acc.at[pl.ds(sid * _RPT, _RPT)],
                        out_hbm.at[pl.ds(base, _RPT)])


def _sc_deg(e0, e1, e2, ones1d, zeros1d):
    mesh = plsc.VectorSubcoreMesh(core_axis_name="c", subcore_axis_name="s")
    return pl.kernel(
        _deg_body,
        out_type=jax.ShapeDtypeStruct((_R * 2 * _NP,), jnp.float32),
        mesh=mesh,
        scratch_types=[
            pltpu.VMEM((_CHUNK,), jnp.float32),
            pltpu.VMEM((_CPT, _CHUNK), jnp.int32),
            pltpu.VMEM_SHARED((_NP,), jnp.float32),
            pltpu.SemaphoreType.DMA,
        ],
    )(e0, e1, e2, ones1d, zeros1d)


def _spmm_body(y0_hbm, y1_hbm, y2_hbm, e0_hbm, e1_hbm, e2_hbm,
               zeros_hbm, out_hbm,
               idxv, buf_a, buf_b, acc, sem_a, sem_b):
    cid = lax.axis_index("c")
    sid = lax.axis_index("s")
    wid = sid * 2 + cid
    ys = [y0_hbm, y1_hbm, y2_hbm]
    edges = [e0_hbm, e1_hbm, e2_hbm]
    pltpu.sync_copy(zeros_hbm, acc.at[pl.ds(sid * _RPT, _RPT)])
    plsc.subcore_barrier()
    for r in range(_R):
        y_hbm = ys[r]
        for h in range(2):
            pltpu.sync_copy(edges[r].at[0, wid, h], idxv.at[1])
            pltpu.sync_copy(edges[r].at[1, wid, h], idxv.at[0])
            colv = idxv.at[0]
            rowv = idxv.at[1]

            pltpu.async_copy(y_hbm.at[colv.at[0]], buf_a, sem_a)

            def pair(k, carry):
                j0 = 2 * k
                pltpu.async_copy(y_hbm.at[colv.at[j0 + 1]], buf_b, sem_b)
                pltpu.make_async_copy(
                    y_hbm.at[colv.at[j0]], buf_a, sem_a).wait()
                pltpu.sync_copy(buf_a, acc.at[rowv.at[j0]], add=True)
                pltpu.async_copy(y_hbm.at[colv.at[j0 + 2]], buf_a, sem_a)
                pltpu.make_async_copy(
                    y_hbm.at[colv.at[j0 + 1]], buf_b, sem_b).wait()
                pltpu.sync_copy(buf_b, acc.at[rowv.at[j0 + 1]], add=True)
                return carry

            lax.fori_loop(0, _HCPT // 2 - 1, pair, 0)
            jlast = _HCPT - 2
            pltpu.async_copy(y_hbm.at[colv.at[jlast + 1]], buf_b, sem_b)
            pltpu.make_async_copy(
                y_hbm.at[colv.at[jlast]], buf_a, sem_a).wait()
            pltpu.sync_copy(buf_a, acc.at[rowv.at[jlast]], add=True)
            pltpu.make_async_copy(
                y_hbm.at[colv.at[jlast + 1]], buf_b, sem_b).wait()
            pltpu.sync_copy(buf_b, acc.at[rowv.at[jlast + 1]], add=True)

        plsc.subcore_barrier()
        pltpu.sync_copy(acc.at[pl.ds(sid * _RPT, _RPT)],
                        out_hbm.at[r, cid, pl.ds(sid * _RPT, _RPT)])
        if r + 1 < _R:
            plsc.subcore_barrier()


def _sc_spmm(y3, e0, e1, e2, zeros128):
    mesh = plsc.VectorSubcoreMesh(core_axis_name="c", subcore_axis_name="s")
    return pl.kernel(
        _spmm_body,
        out_type=jax.ShapeDtypeStruct((_R, 2, _NP, _D), jnp.float32),
        mesh=mesh,
        scratch_types=[
            pltpu.VMEM((2, _HCPT, _CHUNK), jnp.int32),
            pltpu.VMEM((_CHUNK, _D), jnp.float32),
            pltpu.VMEM((_CHUNK, _D), jnp.float32),
            pltpu.VMEM_SHARED((_NP, _D), jnp.float32),
            pltpu.SemaphoreType.DMA,
            pltpu.SemaphoreType.DMA,
        ],
    )(y3[0], y3[1], y3[2], e0, e1, e2, zeros128)


def _prep_body(x_ref, degp_ref, y1_ref, dinv_ref):
    x = x_ref[...]
    dinvs = []
    for r in range(_R):
        deg = degp_ref[:, 2 * r:2 * r + 1] + degp_ref[:, 2 * r + 1:2 * r + 2]
        dinv = jnp.where(deg > 0, lax.rsqrt(jnp.maximum(deg, 1e-12)), 0.0)
        y1_ref[r] = dinv * x
        dinvs.append(dinv)
    dinv_ref[...] = jnp.concatenate(dinvs, axis=1)


def _tc_prep(X, degp):
    return pl.pallas_call(
        _prep_body,
        grid=(_N // _BLK,),
        in_specs=[
            pl.BlockSpec((_BLK, _D), lambda i: (i, 0)),
            pl.BlockSpec((_BLK, 2 * _R), lambda i: (i, 0)),
        ],
        out_specs=[
            pl.BlockSpec((_R, _BLK, _D), lambda i: (0, i, 0)),
            pl.BlockSpec((_BLK, _R), lambda i: (i, 0)),
        ],
        out_shape=[
            jax.ShapeDtypeStruct((_R, _NP, _D), jnp.float32),
            jax.ShapeDtypeStruct((_NP, _R), jnp.float32),
        ],
    )(X, degp)


def _mid_body(p_ref, dinv_ref, w1_ref, y2_ref):
    acc = jnp.zeros((_BLK, _D), jnp.float32)
    prev = None
    for r in range(_R):
        cum = p_ref[r, 0] + p_ref[r, 1]
        seg = cum if prev is None else cum - prev
        prev = cum
        z = dinv_ref[:, r:r + 1] * seg
        acc += jax.nn.relu(jnp.dot(z, w1_ref[r],
                                   preferred_element_type=jnp.float32))
    h1 = acc * (1.0 / _R)
    for r in range(_R):
        y2_ref[r] = dinv_ref[:, r:r + 1] * h1


def _tc_mid(p1, dinv, W1):
    return pl.pallas_call(
        _mid_body,
        grid=(_N // _BLK,),
        in_specs=[
            pl.BlockSpec((_R, 2, _BLK, _D), lambda i: (0, 0, i, 0)),
            pl.BlockSpec((_BLK, _R), lambda i: (i, 0)),
            pl.BlockSpec((_R, _D, _D), lambda i: (0, 0, 0)),
        ],
        out_specs=pl.BlockSpec((_R, _BLK, _D), lambda i: (0, i, 0)),
        out_shape=jax.ShapeDtypeStruct((_R, _NP, _D), jnp.float32),
    )(p1, dinv, W1)


def _out_body(p_ref, dinv_ref, w2_ref, q_ref, tau_ref, ow_ref, ob_ref,
              logits_ref, alpha_ref):
    hs = []
    ss = []
    q = q_ref[...]
    prev = None
    for r in range(_R):
        cum = p_ref[r, 0] + p_ref[r, 1]
        seg = cum if prev is None else cum - prev
        prev = cum
        z = dinv_ref[:, r:r + 1] * seg
        h = jax.nn.relu(jnp.dot(z, w2_ref[r],
                                preferred_element_type=jnp.float32))
        hs.append(h)
        ss.append(jnp.sum(h * q, axis=1, keepdims=True))
    tau_c = jnp.clip(tau_ref[0, 0], 0.5, 5.0)
    m = jnp.maximum(jnp.maximum(ss[0], ss[1]), ss[2])
    es = [jnp.exp((s - m) / tau_c) for s in ss]
    den = es[0] + es[1] + es[2]
    alphas = [e / den for e in es]
    h2 = (hs[0] + hs[1] + hs[2]) * (1.0 / _R)
    for r in range(_R):
        h2 = h2 + alphas[r] * hs[r]
    logits_ref[...] = (jnp.dot(h2, ow_ref[...],
                               preferred_element_type=jnp.float32)
                       + ob_ref[...])
    alpha_ref[...] = jnp.concatenate(alphas, axis=1)


def _tc_out(p2, dinv, W2, q2d, tau2d, out_W, ob2d):
    return pl.pallas_call(
        _out_body,
        grid=(_N // _BLK,),
        in_specs=[
            pl.BlockSpec((_R, 2, _BLK, _D), lambda i: (0, 0, i, 0)),
            pl.BlockSpec((_BLK, _R), lambda i: (i, 0)),
            pl.BlockSpec((_R, _D, _D), lambda i: (0, 0, 0)),
            pl.BlockSpec((1, _D), lambda i: (0, 0)),
            pl.BlockSpec((1, 1), lambda i: (0, 0)),
            pl.BlockSpec((_D, _C), lambda i: (0, 0)),
            pl.BlockSpec((1, _C), lambda i: (0, 0)),
        ],
        out_specs=[
            pl.BlockSpec((_BLK, _C), lambda i: (i, 0)),
            pl.BlockSpec((_BLK, _R), lambda i: (i, 0)),
        ],
        out_shape=[
            jax.ShapeDtypeStruct((_N, _C), jnp.float32),
            jax.ShapeDtypeStruct((_N, _R), jnp.float32),
        ],
    )(p2, dinv, W2, q2d, tau2d, out_W, ob2d)


@jax.jit
def kernel(X, edge_index_r0, edge_index_r1, edge_index_r2, W1, W2,
           att_q, tau, out_W, out_b):
    e0 = edge_index_r0.reshape(2, _NW, 2, _HCPT, _CHUNK)
    e1 = edge_index_r1.reshape(2, _NW, 2, _HCPT, _CHUNK)
    e2 = edge_index_r2.reshape(2, _NW, 2, _HCPT, _CHUNK)

    ones1d = jnp.ones((_CHUNK,), jnp.float32)
    zeros1d = jnp.zeros((_RPT,), jnp.float32)
    zeros128 = jnp.zeros((_RPT, _D), jnp.float32)

    degp = _sc_deg(e0, e1, e2, ones1d, zeros1d).reshape(2 * _R, _NP).T
    y1, dinv = _tc_prep(X, degp)
    p1 = _sc_spmm(y1, e0, e1, e2, zeros128)
    y2 = _tc_mid(p1, dinv, W1)
    p2 = _sc_spmm(y2, e0, e1, e2, zeros128)
    logits, alpha = _tc_out(
        p2, dinv, W2,
        att_q.reshape(1, _D),
        tau.reshape(1, 1),
        out_W,
        out_b.reshape(1, _C),
    )
    return logits, alpha


# speedup vs baseline: 28.3135x; 1.0248x over previous
import functools

import jax
import jax.numpy as jnp
from jax import lax
from jax.experimental import pallas as pl
from jax.experimental.pallas import tpu as pltpu
from jax.experimental.pallas import tpu_sc as plsc

_N = 10000
_E = 320000
_R = 3
_D = 128
_C = 16

_CHUNK = 125
_NW = 32
_CPT = _E // (_CHUNK * _NW)
_HCPT = _CPT // 2
_NP = 10240
_RPT = _NP // 16

_BLK = 1000


def _deg_body(e0_hbm, e1_hbm, e2_hbm, ones_hbm, zeros_hbm, out_hbm,
              onesv, rowv, acc, sem_s):
    cid = lax.axis_index("c")
    sid = lax.axis_index("s")
    wid = sid * 2 + cid
    pltpu.sync_copy(ones_hbm, onesv)
    edges = [e0_hbm, e1_hbm, e2_hbm]
    for r in range(_R):
        pltpu.sync_copy(zeros_hbm, acc.at[pl.ds(sid * _RPT, _RPT)])
        plsc.subcore_barrier()
        for h in range(2):
            pltpu.sync_copy(edges[r].at[0, wid, h],
                            rowv.at[pl.ds(h * _HCPT, _HCPT)])

        for jj in range(4):
            pltpu.async_copy(onesv, acc.at[rowv.at[jj]], sem_s, add=True)

        def chunk(j, carry):
            pltpu.make_async_copy(onesv, acc.at[rowv.at[j]], sem_s).wait()
            pltpu.async_copy(onesv, acc.at[rowv.at[j + 4]], sem_s, add=True)
            return carry

        lax.fori_loop(0, _CPT - 4, chunk, 0)
        for jj in range(_CPT - 4, _CPT):
            pltpu.make_async_copy(onesv, acc.at[rowv.at[jj]], sem_s).wait()
        plsc.subcore_barrier()
        base = pl.multiple_of((r * 2 + cid) * _NP + sid * _RPT, 128)
        pltpu.sync_copy(acc.at[pl.ds(sid * _RPT, _RPT)],
                        out_hbm.at[pl.ds(base, _RPT)])


def _sc_deg(e0, e1, e2, ones1d, zeros1d):
    mesh = plsc.VectorSubcoreMesh(core_axis_name="c", subcore_axis_name="s")
    return pl.kernel(
        _deg_body,
        out_type=jax.ShapeDtypeStruct((_R * 2 * _NP,), jnp.float32),
        mesh=mesh,
        scratch_types=[
            pltpu.VMEM((_CHUNK,), jnp.float32),
            pltpu.VMEM((_CPT, _CHUNK), jnp.int32),
            pltpu.VMEM_SHARED((_NP,), jnp.float32),
            pltpu.SemaphoreType.DMA,
        ],
    )(e0, e1, e2, ones1d, zeros1d)


def _spmm_body(y0_hbm, y1_hbm, y2_hbm, e0_hbm, e1_hbm, e2_hbm,
               zeros_hbm, out_hbm,
               idxv, buf_a, buf_b, acc, sem_a, sem_b):
    cid = lax.axis_index("c")
    sid = lax.axis_index("s")
    wid = sid * 2 + cid
    ys = [y0_hbm, y1_hbm, y2_hbm]
    edges = [e0_hbm, e1_hbm, e2_hbm]
    pltpu.sync_copy(zeros_hbm, acc.at[pl.ds(sid * _RPT, _RPT)])
    plsc.subcore_barrier()
    for r in range(_R):
        y_hbm = ys[r]
        for h in range(2):
            pltpu.sync_copy(edges[r].at[0, wid, h], idxv.at[1])
            pltpu.sync_copy(edges[r].at[1, wid, h], idxv.at[0])
            colv = idxv.at[0]
            rowv = idxv.at[1]

            pltpu.async_copy(y_hbm.at[colv.at[0]], buf_a, sem_a)

            def pair(k, carry):
                j0 = 2 * k
                pltpu.async_copy(y_hbm.at[colv.at[j0 + 1]], buf_b, sem_b)
                pltpu.make_async_copy(
                    y_hbm.at[colv.at[j0]], buf_a, sem_a).wait()
                pltpu.sync_copy(buf_a, acc.at[rowv.at[j0]], add=True)
                pltpu.async_copy(y_hbm.at[colv.at[j0 + 2]], buf_a, sem_a)
                pltpu.make_async_copy(
                    y_hbm.at[colv.at[j0 + 1]], buf_b, sem_b).wait()
                pltpu.sync_copy(buf_b, acc.at[rowv.at[j0 + 1]], add=True)
                return carry

            lax.fori_loop(0, _HCPT // 2 - 1, pair, 0)
            jlast = _HCPT - 2
            pltpu.async_copy(y_hbm.at[colv.at[jlast + 1]], buf_b, sem_b)
            pltpu.make_async_copy(
                y_hbm.at[colv.at[jlast]], buf_a, sem_a).wait()
            pltpu.sync_copy(buf_a, acc.at[rowv.at[jlast]], add=True)
            pltpu.make_async_copy(
                y_hbm.at[colv.at[jlast + 1]], buf_b, sem_b).wait()
            pltpu.sync_copy(buf_b, acc.at[rowv.at[jlast + 1]], add=True)

        plsc.subcore_barrier()
        pltpu.sync_copy(acc.at[pl.ds(sid * _RPT, _RPT)],
                        out_hbm.at[r, cid, pl.ds(sid * _RPT, _RPT)])
        if r + 1 < _R:
            plsc.subcore_barrier()


def _sc_spmm(y3, e0, e1, e2, zeros128):
    mesh = plsc.VectorSubcoreMesh(core_axis_name="c", subcore_axis_name="s")
    return pl.kernel(
        _spmm_body,
        out_type=jax.ShapeDtypeStruct((_R, 2, _NP, _D), jnp.float32),
        mesh=mesh,
        scratch_types=[
            pltpu.VMEM((2, _HCPT, _CHUNK), jnp.int32),
            pltpu.VMEM((_CHUNK, _D), jnp.float32),
            pltpu.VMEM((_CHUNK, _D), jnp.float32),
            pltpu.VMEM_SHARED((_NP, _D), jnp.float32),
            pltpu.SemaphoreType.DMA,
            pltpu.SemaphoreType.DMA,
        ],
    )(y3[0], y3[1], y3[2], e0, e1, e2, zeros128)


def _prep_body(x_ref, degp_ref, ya_ref, yb_ref, yc_ref, dinv_ref):
    x = x_ref[...]
    y_refs = [ya_ref, yb_ref, yc_ref]
    dinvs = []
    for r in range(_R):
        deg = degp_ref[:, 2 * r:2 * r + 1] + degp_ref[:, 2 * r + 1:2 * r + 2]
        dinv = jnp.where(deg > 0, lax.rsqrt(jnp.maximum(deg, 1e-12)), 0.0)
        y_refs[r][...] = dinv * x
        dinvs.append(dinv)
    dinv_ref[...] = jnp.concatenate(dinvs, axis=1)


def _tc_prep(X, degp):
    return pl.pallas_call(
        _prep_body,
        grid=(_N // _BLK,),
        in_specs=[
            pl.BlockSpec((_BLK, _D), lambda i: (i, 0)),
            pl.BlockSpec((_BLK, 2 * _R), lambda i: (i, 0)),
        ],
        out_specs=[
            pl.BlockSpec((_BLK, _D), lambda i: (i, 0)),
            pl.BlockSpec((_BLK, _D), lambda i: (i, 0)),
            pl.BlockSpec((_BLK, _D), lambda i: (i, 0)),
            pl.BlockSpec((_BLK, _R), lambda i: (i, 0)),
        ],
        out_shape=[
            jax.ShapeDtypeStruct((_NP, _D), jnp.float32),
            jax.ShapeDtypeStruct((_NP, _D), jnp.float32),
            jax.ShapeDtypeStruct((_NP, _D), jnp.float32),
            jax.ShapeDtypeStruct((_BLK * (_N // _BLK), _R), jnp.float32),
        ],
    )(X, degp)


def _mid_body(p_ref, dinv_ref, w1_ref, ya_ref, yb_ref, yc_ref):
    acc = jnp.zeros((_BLK, _D), jnp.float32)
    prev = None
    for r in range(_R):
        cum = p_ref[r, 0] + p_ref[r, 1]
        seg = cum if prev is None else cum - prev
        prev = cum
        z = dinv_ref[:, r:r + 1] * seg
        acc += jax.nn.relu(jnp.dot(z, w1_ref[r],
                                   preferred_element_type=jnp.float32))
    h1 = acc * (1.0 / _R)
    y_refs = [ya_ref, yb_ref, yc_ref]
    for r in range(_R):
        y_refs[r][...] = dinv_ref[:, r:r + 1] * h1


def _tc_mid(p1, dinv, W1):
    return pl.pallas_call(
        _mid_body,
        grid=(_N // _BLK,),
        in_specs=[
            pl.BlockSpec((_R, 2, _BLK, _D), lambda i: (0, 0, i, 0)),
            pl.BlockSpec((_BLK, _R), lambda i: (i, 0)),
            pl.BlockSpec((_R, _D, _D), lambda i: (0, 0, 0)),
        ],
        out_specs=[
            pl.BlockSpec((_BLK, _D), lambda i: (i, 0)),
            pl.BlockSpec((_BLK, _D), lambda i: (i, 0)),
            pl.BlockSpec((_BLK, _D), lambda i: (i, 0)),
        ],
        out_shape=[
            jax.ShapeDtypeStruct((_NP, _D), jnp.float32),
            jax.ShapeDtypeStruct((_NP, _D), jnp.float32),
            jax.ShapeDtypeStruct((_NP, _D), jnp.float32),
        ],
    )(p1, dinv, W1)


def _out_body(p_ref, dinv_ref, w2_ref, q_ref, tau_ref, ow_ref, ob_ref,
              logits_ref, alpha_ref):
    hs = []
    ss = []
    q = q_ref[...]
    prev = None
    for r in range(_R):
        cum = p_ref[r, 0] + p_ref[r, 1]
        seg = cum if prev is None else cum - prev
        prev = cum
        z = dinv_ref[:, r:r + 1] * seg
        h = jax.nn.relu(jnp.dot(z, w2_ref[r],
                                preferred_element_type=jnp.float32))
        hs.append(h)
        ss.append(jnp.sum(h * q, axis=1, keepdims=True))
    tau_c = jnp.clip(tau_ref[0, 0], 0.5, 5.0)
    m = jnp.maximum(jnp.maximum(ss[0], ss[1]), ss[2])
    es = [jnp.exp((s - m) / tau_c) for s in ss]
    den = es[0] + es[1] + es[2]
    alphas = [e / den for e in es]
    h2 = (hs[0] + hs[1] + hs[2]) * (1.0 / _R)
    for r in range(_R):
        h2 = h2 + alphas[r] * hs[r]
    logits_ref[...] = (jnp.dot(h2, ow_ref[...],
                               preferred_element_type=jnp.float32)
                       + ob_ref[...])
    alpha_ref[...] = jnp.concatenate(alphas, axis=1)


def _tc_out(p2, dinv, W2, q2d, tau2d, out_W, ob2d):
    return pl.pallas_call(
        _out_body,
        grid=(_N // _BLK,),
        in_specs=[
            pl.BlockSpec((_R, 2, _BLK, _D), lambda i: (0, 0, i, 0)),
            pl.BlockSpec((_BLK, _R), lambda i: (i, 0)),
            pl.BlockSpec((_R, _D, _D), lambda i: (0, 0, 0)),
            pl.BlockSpec((1, _D), lambda i: (0, 0)),
            pl.BlockSpec((1, 1), lambda i: (0, 0)),
            pl.BlockSpec((_D, _C), lambda i: (0, 0)),
            pl.BlockSpec((1, _C), lambda i: (0, 0)),
        ],
        out_specs=[
            pl.BlockSpec((_BLK, _C), lambda i: (i, 0)),
            pl.BlockSpec((_BLK, _R), lambda i: (i, 0)),
        ],
        out_shape=[
            jax.ShapeDtypeStruct((_N, _C), jnp.float32),
            jax.ShapeDtypeStruct((_N, _R), jnp.float32),
        ],
    )(p2, dinv, W2, q2d, tau2d, out_W, ob2d)


@jax.jit
def kernel(X, edge_index_r0, edge_index_r1, edge_index_r2, W1, W2,
           att_q, tau, out_W, out_b):
    e0 = edge_index_r0.reshape(2, _NW, 2, _HCPT, _CHUNK)
    e1 = edge_index_r1.reshape(2, _NW, 2, _HCPT, _CHUNK)
    e2 = edge_index_r2.reshape(2, _NW, 2, _HCPT, _CHUNK)

    ones1d = jnp.ones((_CHUNK,), jnp.float32)
    zeros1d = jnp.zeros((_RPT,), jnp.float32)
    zeros128 = jnp.zeros((_RPT, _D), jnp.float32)

    degp = _sc_deg(e0, e1, e2, ones1d, zeros1d).reshape(2 * _R, _NP).T
    y1a, y1b, y1c, dinv = _tc_prep(X, degp)
    p1 = _sc_spmm((y1a, y1b, y1c), e0, e1, e2, zeros128)
    y2a, y2b, y2c = _tc_mid(p1, dinv, W1)
    p2 = _sc_spmm((y2a, y2b, y2c), e0, e1, e2, zeros128)
    logits, alpha = _tc_out(
        p2, dinv, W2,
        att_q.reshape(1, _D),
        tau.reshape(1, 1),
        out_W,
        out_b.reshape(1, _C),
    )
    return logits, alpha


# speedup vs baseline: 28.6378x; 1.0115x over previous
import functools

import jax
import jax.numpy as jnp
from jax import lax
from jax.experimental import pallas as pl
from jax.experimental.pallas import tpu as pltpu
from jax.experimental.pallas import tpu_sc as plsc

_N = 10000
_E = 320000
_R = 3
_D = 128
_C = 16

_CHUNK = 125
_NW = 32
_CPT = _E // (_CHUNK * _NW)
_HCPT = _CPT // 2
_NP = 10240
_RPT = _NP // 16

_BLK = 1000


def _deg_body(e0_hbm, e1_hbm, e2_hbm, ones_hbm, zeros_hbm, out_hbm,
              onesv, rowv, acc, sem_s):
    cid = lax.axis_index("c")
    sid = lax.axis_index("s")
    wid = sid * 2 + cid
    pltpu.sync_copy(ones_hbm, onesv)
    edges = [e0_hbm, e1_hbm, e2_hbm]
    for r in range(_R):
        pltpu.sync_copy(zeros_hbm, acc.at[pl.ds(sid * _RPT, _RPT)])
        plsc.subcore_barrier()
        for h in range(2):
            pltpu.sync_copy(edges[r].at[0, wid, h],
                            rowv.at[pl.ds(h * _HCPT, _HCPT)])

        for jj in range(4):
            pltpu.async_copy(onesv, acc.at[rowv.at[jj]], sem_s, add=True)

        def chunk(j, carry):
            pltpu.make_async_copy(onesv, acc.at[rowv.at[j]], sem_s).wait()
            pltpu.async_copy(onesv, acc.at[rowv.at[j + 4]], sem_s, add=True)
            return carry

        lax.fori_loop(0, _CPT - 4, chunk, 0)
        for jj in range(_CPT - 4, _CPT):
            pltpu.make_async_copy(onesv, acc.at[rowv.at[jj]], sem_s).wait()
        plsc.subcore_barrier()
        base = pl.multiple_of((r * 2 + cid) * _NP + sid * _RPT, 128)
        pltpu.sync_copy(acc.at[pl.ds(sid * _RPT, _RPT)],
                        out_hbm.at[pl.ds(base, _RPT)])


def _sc_deg(e0, e1, e2, ones1d, zeros1d):
    mesh = plsc.VectorSubcoreMesh(core_axis_name="c", subcore_axis_name="s")
    return pl.kernel(
        _deg_body,
        out_type=jax.ShapeDtypeStruct((_R * 2 * _NP,), jnp.float32),
        mesh=mesh,
        scratch_types=[
            pltpu.VMEM((_CHUNK,), jnp.float32),
            pltpu.VMEM((_CPT, _CHUNK), jnp.int32),
            pltpu.VMEM_SHARED((_NP,), jnp.float32),
            pltpu.SemaphoreType.DMA,
        ],
    )(e0, e1, e2, ones1d, zeros1d)


def _spmm_body(y0_hbm, y1_hbm, y2_hbm, e0_hbm, e1_hbm, e2_hbm,
               zeros_hbm, out_hbm,
               idxv, buf_a, buf_b, acc, sem_a, sem_b):
    cid = lax.axis_index("c")
    sid = lax.axis_index("s")
    wid = sid * 2 + cid
    ys = [y0_hbm, y1_hbm, y2_hbm]
    edges = [e0_hbm, e1_hbm, e2_hbm]
    segs = [(r, h) for r in range(_R) for h in range(2)]

    def load_idx(r, h):
        pltpu.sync_copy(edges[r].at[0, wid, h], idxv.at[1])
        pltpu.sync_copy(edges[r].at[1, wid, h], idxv.at[0])

    colv = idxv.at[0]
    rowv = idxv.at[1]

    load_idx(0, 0)
    pltpu.async_copy(ys[0].at[colv.at[0]], buf_a, sem_a)
    pltpu.sync_copy(zeros_hbm, acc.at[pl.ds(sid * _RPT, _RPT)])
    plsc.subcore_barrier()

    for i, (r, h) in enumerate(segs):
        y_hbm = ys[r]

        def pair(k, carry):
            j0 = 2 * k
            pltpu.async_copy(y_hbm.at[colv.at[j0 + 1]], buf_b, sem_b)
            pltpu.make_async_copy(
                y_hbm.at[colv.at[j0]], buf_a, sem_a).wait()
            pltpu.sync_copy(buf_a, acc.at[rowv.at[j0]], add=True)
            pltpu.async_copy(y_hbm.at[colv.at[j0 + 2]], buf_a, sem_a)
            pltpu.make_async_copy(
                y_hbm.at[colv.at[j0 + 1]], buf_b, sem_b).wait()
            pltpu.sync_copy(buf_b, acc.at[rowv.at[j0 + 1]], add=True)
            return carry

        lax.fori_loop(0, _HCPT // 2 - 1, pair, 0)
        jlast = _HCPT - 2
        pltpu.async_copy(y_hbm.at[colv.at[jlast + 1]], buf_b, sem_b)
        pltpu.make_async_copy(y_hbm.at[colv.at[jlast]], buf_a, sem_a).wait()
        pltpu.sync_copy(buf_a, acc.at[rowv.at[jlast]], add=True)
        pltpu.make_async_copy(
            y_hbm.at[colv.at[jlast + 1]], buf_b, sem_b).wait()
        pltpu.sync_copy(buf_b, acc.at[rowv.at[jlast + 1]], add=True)

        if i + 1 < len(segs):
            rn, hn = segs[i + 1]
            load_idx(rn, hn)
            pltpu.async_copy(ys[rn].at[colv.at[0]], buf_a, sem_a)

        if h == 1:
            plsc.subcore_barrier()
            pltpu.sync_copy(acc.at[pl.ds(sid * _RPT, _RPT)],
                            out_hbm.at[r, cid, pl.ds(sid * _RPT, _RPT)])
            if r + 1 < _R:
                plsc.subcore_barrier()


def _sc_spmm(y3, e0, e1, e2, zeros128):
    mesh = plsc.VectorSubcoreMesh(core_axis_name="c", subcore_axis_name="s")
    return pl.kernel(
        _spmm_body,
        out_type=jax.ShapeDtypeStruct((_R, 2, _NP, _D), jnp.float32),
        mesh=mesh,
        scratch_types=[
            pltpu.VMEM((2, _HCPT, _CHUNK), jnp.int32),
            pltpu.VMEM((_CHUNK, _D), jnp.float32),
            pltpu.VMEM((_CHUNK, _D), jnp.float32),
            pltpu.VMEM_SHARED((_NP, _D), jnp.float32),
            pltpu.SemaphoreType.DMA,
            pltpu.SemaphoreType.DMA,
        ],
    )(y3[0], y3[1], y3[2], e0, e1, e2, zeros128)


def _prep_body(x_ref, degp_ref, ya_ref, yb_ref, yc_ref, dinv_ref):
    x = x_ref[...]
    y_refs = [ya_ref, yb_ref, yc_ref]
    dinvs = []
    for r in range(_R):
        deg = degp_ref[:, 2 * r:2 * r + 1] + degp_ref[:, 2 * r + 1:2 * r + 2]
        dinv = jnp.where(deg > 0, lax.rsqrt(jnp.maximum(deg, 1e-12)), 0.0)
        y_refs[r][...] = dinv * x
        dinvs.append(dinv)
    dinv_ref[...] = jnp.concatenate(dinvs, axis=1)


def _tc_prep(X, degp):
    return pl.pallas_call(
        _prep_body,
        grid=(_N // _BLK,),
        in_specs=[
            pl.BlockSpec((_BLK, _D), lambda i: (i, 0)),
            pl.BlockSpec((_BLK, 2 * _R), lambda i: (i, 0)),
        ],
        out_specs=[
            pl.BlockSpec((_BLK, _D), lambda i: (i, 0)),
            pl.BlockSpec((_BLK, _D), lambda i: (i, 0)),
            pl.BlockSpec((_BLK, _D), lambda i: (i, 0)),
            pl.BlockSpec((_BLK, _R), lambda i: (i, 0)),
        ],
        out_shape=[
            jax.ShapeDtypeStruct((_NP, _D), jnp.float32),
            jax.ShapeDtypeStruct((_NP, _D), jnp.float32),
            jax.ShapeDtypeStruct((_NP, _D), jnp.float32),
            jax.ShapeDtypeStruct((_BLK * (_N // _BLK), _R), jnp.float32),
        ],
    )(X, degp)


def _mid_body(p_ref, dinv_ref, w1_ref, ya_ref, yb_ref, yc_ref):
    acc = jnp.zeros((_BLK, _D), jnp.float32)
    prev = None
    for r in range(_R):
        cum = p_ref[r, 0] + p_ref[r, 1]
        seg = cum if prev is None else cum - prev
        prev = cum
        z = dinv_ref[:, r:r + 1] * seg
        acc += jax.nn.relu(jnp.dot(z, w1_ref[r],
                                   preferred_element_type=jnp.float32))
    h1 = acc * (1.0 / _R)
    y_refs = [ya_ref, yb_ref, yc_ref]
    for r in range(_R):
        y_refs[r][...] = dinv_ref[:, r:r + 1] * h1


def _tc_mid(p1, dinv, W1):
    return pl.pallas_call(
        _mid_body,
        grid=(_N // _BLK,),
        in_specs=[
            pl.BlockSpec((_R, 2, _BLK, _D), lambda i: (0, 0, i, 0)),
            pl.BlockSpec((_BLK, _R), lambda i: (i, 0)),
            pl.BlockSpec((_R, _D, _D), lambda i: (0, 0, 0)),
        ],
        out_specs=[
            pl.BlockSpec((_BLK, _D), lambda i: (i, 0)),
            pl.BlockSpec((_BLK, _D), lambda i: (i, 0)),
            pl.BlockSpec((_BLK, _D), lambda i: (i, 0)),
        ],
        out_shape=[
            jax.ShapeDtypeStruct((_NP, _D), jnp.float32),
            jax.ShapeDtypeStruct((_NP, _D), jnp.float32),
            jax.ShapeDtypeStruct((_NP, _D), jnp.float32),
        ],
    )(p1, dinv, W1)


def _out_body(p_ref, dinv_ref, w2_ref, q_ref, tau_ref, ow_ref, ob_ref,
              logits_ref, alpha_ref):
    hs = []
    ss = []
    q = q_ref[...]
    prev = None
    for r in range(_R):
        cum = p_ref[r, 0] + p_ref[r, 1]
        seg = cum if prev is None else cum - prev
        prev = cum
        z = dinv_ref[:, r:r + 1] * seg
        h = jax.nn.relu(jnp.dot(z, w2_ref[r],
                                preferred_element_type=jnp.float32))
        hs.append(h)
        ss.append(jnp.sum(h * q, axis=1, keepdims=True))
    tau_c = jnp.clip(tau_ref[0, 0], 0.5, 5.0)
    m = jnp.maximum(jnp.maximum(ss[0], ss[1]), ss[2])
    es = [jnp.exp((s - m) / tau_c) for s in ss]
    den = es[0] + es[1] + es[2]
    alphas = [e / den for e in es]
    h2 = (hs[0] + hs[1] + hs[2]) * (1.0 / _R)
    for r in range(_R):
        h2 = h2 + alphas[r] * hs[r]
    logits_ref[...] = (jnp.dot(h2, ow_ref[...],
                               preferred_element_type=jnp.float32)
                       + ob_ref[...])
    alpha_ref[...] = jnp.concatenate(alphas, axis=1)


def _tc_out(p2, dinv, W2, q2d, tau2d, out_W, ob2d):
    return pl.pallas_call(
        _out_body,
        grid=(_N // _BLK,),
        in_specs=[
            pl.BlockSpec((_R, 2, _BLK, _D), lambda i: (0, 0, i, 0)),
            pl.BlockSpec((_BLK, _R), lambda i: (i, 0)),
            pl.BlockSpec((_R, _D, _D), lambda i: (0, 0, 0)),
            pl.BlockSpec((1, _D), lambda i: (0, 0)),
            pl.BlockSpec((1, 1), lambda i: (0, 0)),
            pl.BlockSpec((_D, _C), lambda i: (0, 0)),
            pl.BlockSpec((1, _C), lambda i: (0, 0)),
        ],
        out_specs=[
            pl.BlockSpec((_BLK, _C), lambda i: (i, 0)),
            pl.BlockSpec((_BLK, _R), lambda i: (i, 0)),
        ],
        out_shape=[
            jax.ShapeDtypeStruct((_N, _C), jnp.float32),
            jax.ShapeDtypeStruct((_N, _R), jnp.float32),
        ],
    )(p2, dinv, W2, q2d, tau2d, out_W, ob2d)


@jax.jit
def kernel(X, edge_index_r0, edge_index_r1, edge_index_r2, W1, W2,
           att_q, tau, out_W, out_b):
    e0 = edge_index_r0.reshape(2, _NW, 2, _HCPT, _CHUNK)
    e1 = edge_index_r1.reshape(2, _NW, 2, _HCPT, _CHUNK)
    e2 = edge_index_r2.reshape(2, _NW, 2, _HCPT, _CHUNK)

    ones1d = jnp.ones((_CHUNK,), jnp.float32)
    zeros1d = jnp.zeros((_RPT,), jnp.float32)
    zeros128 = jnp.zeros((_RPT, _D), jnp.float32)

    degp = _sc_deg(e0, e1, e2, ones1d, zeros1d).reshape(2 * _R, _NP).T
    y1a, y1b, y1c, dinv = _tc_prep(X, degp)
    p1 = _sc_spmm((y1a, y1b, y1c), e0, e1, e2, zeros128)
    y2a, y2b, y2c = _tc_mid(p1, dinv, W1)
    p2 = _sc_spmm((y2a, y2b, y2c), e0, e1, e2, zeros128)
    logits, alpha = _tc_out(
        p2, dinv, W2,
        att_q.reshape(1, _D),
        tau.reshape(1, 1),
        out_W,
        out_b.reshape(1, _C),
    )
    return logits, alpha
